# baseline (speedup 1.0000x reference)
import functools

import jax
import jax.numpy as jnp
from jax import lax
from jax.experimental import pallas as pl
from jax.experimental.pallas import tpu as pltpu
from jax.experimental.pallas import tpu_sc as plsc

B = 16384
V = 1_000_000
D = 32
G = 16
CH = 65536
SH = 16
NBLK = (V + CH - 1) // CH
RPB = CH // G
NC, NS, L = 2, 16, 16
NW = NC * NS
NPW = B // NW
NSD = 2 * NPW
C = 64
NCH = NSD // C
NGC = C // L

_mesh = plsc.VectorSubcoreMesh(
    core_axis_name="c", subcore_axis_name="s", num_cores=NC, num_subcores=NS
)


def _tc_restage(t_ref, out_ref):
    out_ref[...] = t_ref[...].reshape(1, D, CH // 128, 128)


def _restage(t3):
    return pl.pallas_call(
        _tc_restage,
        grid=(NBLK,),
        in_specs=[pl.BlockSpec((1, D, CH), lambda i: (0, 0, i))],
        out_specs=pl.BlockSpec((1, D, CH // 128, 128), lambda i: (i, 0, 0, 0)),
        out_shape=jax.ShapeDtypeStruct((NBLK, D, CH // 128, 128), jnp.float32),
        compiler_params=pltpu.CompilerParams(
            dimension_semantics=("parallel",),
        ),
    )(t3)


def _rsqrt(x):
    i = plsc.bitcast(x, jnp.int32)
    i = jnp.int32(0x5F3759DF) - (i >> 1)
    y = plsc.bitcast(i, jnp.float32)
    for _ in range(3):
        y = y * (1.5 - 0.5 * x * y * y)
    return y


@functools.partial(
    pl.kernel,
    out_type=jax.ShapeDtypeStruct((B,), jnp.float32),
    mesh=_mesh,
    compiler_params=pltpu.CompilerParams(
        needs_layout_passes=False, use_tc_tiling_on_sc=False
    ),
    scratch_types=[
        pltpu.VMEM((NSD,), jnp.int32),
        pltpu.VMEM((D * C,), jnp.int32),
        pltpu.VMEM((D * C,), jnp.int32),
        pltpu.VMEM((D * C, G), jnp.float32),
        pltpu.VMEM((D * C, G), jnp.float32),
        pltpu.VMEM((NPW,), jnp.float32),
        pltpu.SemaphoreType.DMA,
        pltpu.SemaphoreType.DMA,
    ],
)
def _cosine_sc(x_hbm, t_hbm, out_hbm, vid_v, idx_a, idx_b, rows_a, rows_b,
               out_v, sem_a, sem_b):
    wid = lax.axis_index("s") * NC + lax.axis_index("c")
    base = wid * NPW
    pltpu.sync_copy(x_hbm.at[pl.ds(2 * base, NSD)], vid_v)
    lane = lax.iota(jnp.int32, L)

    def build(c, idx_ref):
        for g in range(NGC):
            vk = vid_v[pl.ds(c * C + g * L, L)]
            rb = (vk >> SH) * (D * RPB) + ((vk >> 4) & (RPB - 1))
            for d in range(D):
                idx_ref[pl.ds(d * C + g * L, L)] = rb + d * RPB

    def extract(c, rows_ref):
        for g in range(NGC // 2):
            k1 = 32 * g + 2 * lane
            k2 = k1 + 1
            v1 = plsc.load_gather(vid_v, [c * C + k1])
            v2 = plsc.load_gather(vid_v, [c * C + k2])
            c1 = v1 & (G - 1)
            c2 = v2 & (G - 1)
            dot = jnp.zeros((L,), jnp.float32)
            s1 = jnp.zeros((L,), jnp.float32)
            s2 = jnp.zeros((L,), jnp.float32)
            for d in range(D):
                e1 = plsc.load_gather(rows_ref, [d * C + k1, c1])
                e2 = plsc.load_gather(rows_ref, [d * C + k2, c2])
                dot = dot + e1 * e2
                s1 = s1 + e1 * e1
                s2 = s2 + e2 * e2
            sim = dot * _rsqrt(jnp.maximum(s1, 1e-16)) * _rsqrt(
                jnp.maximum(s2, 1e-16))
            out_v[pl.ds(c * (C // 2) + g * L, L)] = 0.5 + 0.5 * sim

    build(0, idx_a)
    pltpu.async_copy(t_hbm.at[idx_a], rows_a, sem_a)

    def two_chunks(i, carry):
        ca = 2 * i
        cb = 2 * i + 1
        build(cb, idx_b)
        pltpu.async_copy(t_hbm.at[idx_b], rows_b, sem_b)
        pltpu.make_async_copy(t_hbm.at[idx_a], rows_a, sem_a).wait()
        extract(ca, rows_a)
        cn = (ca + 2) & (NCH - 1)
        build(cn, idx_a)
        pltpu.async_copy(t_hbm.at[idx_a], rows_a, sem_a)
        pltpu.make_async_copy(t_hbm.at[idx_b], rows_b, sem_b).wait()
        extract(cb, rows_b)
        return carry

    lax.fori_loop(0, NCH // 2, two_chunks, 0)
    pltpu.make_async_copy(t_hbm.at[idx_a], rows_a, sem_a).wait()

    pltpu.sync_copy(out_v, out_hbm.at[pl.ds(base, NPW)])


def kernel(x, table):
    t_lin = _restage(table.T.reshape(1, D, V))
    t16 = t_lin.reshape(NBLK * D * RPB, G)
    return _cosine_sc(x.reshape(-1).astype(jnp.int32), t16)


# speedup vs baseline: 1.0009x; 1.0009x over previous
import functools

import jax
import jax.numpy as jnp
from jax import lax
from jax.experimental import pallas as pl
from jax.experimental.pallas import tpu as pltpu
from jax.experimental.pallas import tpu_sc as plsc

B = 16384
V = 1_000_000
D = 32
G = 16
CH = 32768
SH = 15
NBLK = (V + CH - 1) // CH
RPB = CH // G
NC, NS, L = 2, 16, 16
NW = NC * NS
NPW = B // NW
NSD = 2 * NPW
C = 64
NCH = NSD // C
NGC = C // L

_mesh = plsc.VectorSubcoreMesh(
    core_axis_name="c", subcore_axis_name="s", num_cores=NC, num_subcores=NS
)


def _tc_restage(t_ref, out_ref):
    out_ref[...] = t_ref[...].reshape(1, D, CH // 128, 128)


def _restage(t3):
    return pl.pallas_call(
        _tc_restage,
        grid=(NBLK,),
        in_specs=[pl.BlockSpec((1, D, CH), lambda i: (0, 0, i))],
        out_specs=pl.BlockSpec((1, D, CH // 128, 128), lambda i: (i, 0, 0, 0)),
        out_shape=jax.ShapeDtypeStruct((NBLK, D, CH // 128, 128), jnp.float32),
        compiler_params=pltpu.CompilerParams(
            dimension_semantics=("parallel",),
        ),
    )(t3)


def _rsqrt(x):
    i = plsc.bitcast(x, jnp.int32)
    i = jnp.int32(0x5F3759DF) - (i >> 1)
    y = plsc.bitcast(i, jnp.float32)
    for _ in range(3):
        y = y * (1.5 - 0.5 * x * y * y)
    return y


@functools.partial(
    pl.kernel,
    out_type=jax.ShapeDtypeStruct((B,), jnp.float32),
    mesh=_mesh,
    compiler_params=pltpu.CompilerParams(
        needs_layout_passes=False, use_tc_tiling_on_sc=False
    ),
    scratch_types=[
        pltpu.VMEM((NSD,), jnp.int32),
        pltpu.VMEM((D * C,), jnp.int32),
        pltpu.VMEM((D * C,), jnp.int32),
        pltpu.VMEM((D * C, G), jnp.float32),
        pltpu.VMEM((D * C, G), jnp.float32),
        pltpu.VMEM((NPW,), jnp.float32),
        pltpu.SemaphoreType.DMA,
        pltpu.SemaphoreType.DMA,
    ],
)
def _cosine_sc(x_hbm, t_hbm, out_hbm, vid_v, idx_a, idx_b, rows_a, rows_b,
               out_v, sem_a, sem_b):
    wid = lax.axis_index("s") * NC + lax.axis_index("c")
    base = wid * NPW
    pltpu.sync_copy(x_hbm.at[pl.ds(2 * base, NSD)], vid_v)
    lane = lax.iota(jnp.int32, L)

    def build(c, idx_ref):
        for g in range(NGC):
            vk = vid_v[pl.ds(c * C + g * L, L)]
            rb = (vk >> SH) * (D * RPB) + ((vk >> 4) & (RPB - 1))
            for d in range(D):
                idx_ref[pl.ds(d * C + g * L, L)] = rb + d * RPB

    def extract(c, rows_ref):
        for g in range(NGC // 2):
            k1 = 32 * g + 2 * lane
            k2 = k1 + 1
            v1 = plsc.load_gather(vid_v, [c * C + k1])
            v2 = plsc.load_gather(vid_v, [c * C + k2])
            c1 = v1 & (G - 1)
            c2 = v2 & (G - 1)
            dot = jnp.zeros((L,), jnp.float32)
            s1 = jnp.zeros((L,), jnp.float32)
            s2 = jnp.zeros((L,), jnp.float32)
            for d in range(D):
                e1 = plsc.load_gather(rows_ref, [d * C + k1, c1])
                e2 = plsc.load_gather(rows_ref, [d * C + k2, c2])
                dot = dot + e1 * e2
                s1 = s1 + e1 * e1
                s2 = s2 + e2 * e2
            sim = dot * _rsqrt(jnp.maximum(s1, 1e-16)) * _rsqrt(
                jnp.maximum(s2, 1e-16))
            out_v[pl.ds(c * (C // 2) + g * L, L)] = 0.5 + 0.5 * sim

    build(0, idx_a)
    pltpu.async_copy(t_hbm.at[idx_a], rows_a, sem_a)

    def two_chunks(i, carry):
        ca = 2 * i
        cb = 2 * i + 1
        build(cb, idx_b)
        pltpu.async_copy(t_hbm.at[idx_b], rows_b, sem_b)
        pltpu.make_async_copy(t_hbm.at[idx_a], rows_a, sem_a).wait()
        extract(ca, rows_a)
        cn = (ca + 2) & (NCH - 1)
        build(cn, idx_a)
        pltpu.async_copy(t_hbm.at[idx_a], rows_a, sem_a)
        pltpu.make_async_copy(t_hbm.at[idx_b], rows_b, sem_b).wait()
        extract(cb, rows_b)
        return carry

    lax.fori_loop(0, NCH // 2, two_chunks, 0)
    pltpu.make_async_copy(t_hbm.at[idx_a], rows_a, sem_a).wait()

    pltpu.sync_copy(out_v, out_hbm.at[pl.ds(base, NPW)])


def kernel(x, table):
    t_lin = _restage(table.T.reshape(1, D, V))
    t16 = t_lin.reshape(NBLK * D * RPB, G)
    return _cosine_sc(x.reshape(-1).astype(jnp.int32), t16)
